# baseline (speedup 1.0000x reference)
import functools

import jax
import jax.numpy as jnp
from jax import lax
from jax.experimental import pallas as pl
from jax.experimental.pallas import tpu as pltpu
from jax.experimental.pallas import tpu_sc as plsc

B = 16384
D = 64
KG_L2_LAMBDA = 1e-05
CF_L2_LAMBDA = 1e-05

_NC = 2
_NS = 16
_NW = _NC * _NS
_BPW = B // _NW


def _sc_gather_body(user_ids, item_pos_ids, item_neg_ids, h, r, pos_t, neg_t,
                    user_table, item_table, entity_table, relation_table,
                    out_u, out_ip, out_in, out_ipk, out_ink,
                    out_h, out_r, out_pt, out_nt,
                    idx_v, rows_v, sem):
    wid = lax.axis_index("s") * _NC + lax.axis_index("c")
    base = wid * _BPW

    plan = (
        (user_ids, ((user_table, out_u),)),
        (item_pos_ids, ((item_table, out_ip), (entity_table, out_ipk))),
        (item_neg_ids, ((item_table, out_in), (entity_table, out_ink))),
        (h, ((entity_table, out_h),)),
        (r, ((relation_table, out_r),)),
        (pos_t, ((entity_table, out_pt),)),
        (neg_t, ((entity_table, out_nt),)),
    )
    for idx_hbm, jobs in plan:
        pltpu.sync_copy(idx_hbm.at[pl.ds(base, _BPW)], idx_v)
        for table_hbm, out_hbm in jobs:
            pltpu.async_copy(table_hbm.at[idx_v], rows_v, sem).wait()
            pltpu.sync_copy(rows_v, out_hbm.at[pl.ds(base, _BPW)])


_row_t = jax.ShapeDtypeStruct((B, D), jnp.float32)


@functools.cache
def _sc_gather():
    return pl.kernel(
        _sc_gather_body,
        mesh=plsc.VectorSubcoreMesh(core_axis_name="c", subcore_axis_name="s"),
        out_type=[_row_t] * 9,
        scratch_types=[
            pltpu.VMEM((_BPW,), jnp.int32),
            pltpu.VMEM((_BPW, D), jnp.float32),
            pltpu.SemaphoreType.DMA,
        ],
        compiler_params=pltpu.CompilerParams(use_tc_tiling_on_sc=False),
    )


_BLK = 2048
_NBLK = B // _BLK


def _loss_body(u_ref, ip_ref, in_ref, ipk_ref, ink_ref,
               h_ref, r_ref, pt_ref, nt_ref, out_ref, acc):
    i = pl.program_id(0)

    @pl.when(i == 0)
    def _init():
        for j in range(9):
            acc[j] = 0.0

    @pl.when(i < _NBLK)
    def _phase1():
        hb = h_ref[...]
        rb = r_ref[...]
        pb = pt_ref[...]
        nb = nt_ref[...]
        acc[0] += jnp.sum(hb * hb)
        acc[1] += jnp.sum(rb * rb)
        acc[2] += jnp.sum(pb * pb)
        acc[3] += jnp.sum(nb * nb)
        ub = u_ref[...]
        ipcf = ip_ref[...] + ipk_ref[...]
        incf = in_ref[...] + ink_ref[...]
        x = jnp.sum(ub * (ipcf - incf), axis=1)
        sig = 1.0 / (1.0 + jnp.exp(-x))
        acc[4] += jnp.sum(-jnp.log(1e-10 + sig))
        acc[5] += jnp.sum(ub * ub)
        acc[6] += jnp.sum(ipcf * ipcf)
        acc[7] += jnp.sum(incf * incf)

    @pl.when(i >= _NBLK)
    def _phase2():
        nh = jnp.maximum(jnp.sqrt(acc[0]), 1e-10)
        nr = jnp.maximum(jnp.sqrt(acc[1]), 1e-10)
        np_ = jnp.maximum(jnp.sqrt(acc[2]), 1e-10)
        nn = jnp.maximum(jnp.sqrt(acc[3]), 1e-10)
        hr = h_ref[...] / nh + r_ref[...] / nr
        pos_score = jnp.abs(hr - pt_ref[...] / np_)
        neg_score = jnp.abs(hr - nt_ref[...] / nn)
        y = neg_score - pos_score
        sig = 1.0 / (1.0 + jnp.exp(-y))
        acc[8] += jnp.sum(-jnp.log(1e-09 + sig))

    @pl.when(i == 2 * _NBLK - 1)
    def _final():
        nh2 = jnp.maximum(acc[0], 1e-20)
        nr2 = jnp.maximum(acc[1], 1e-20)
        np2 = jnp.maximum(acc[2], 1e-20)
        nn2 = jnp.maximum(acc[3], 1e-20)
        kg_l2 = (acc[0] / nh2 + acc[1] / nr2 + acc[2] / np2
                 + acc[3] / nn2) / (2.0 * B)
        kg_loss = acc[8] / (B * D)
        cf_loss = acc[4] / B
        cf_l2 = (acc[5] + acc[6] + acc[7]) / (2.0 * B)
        out_ref[0, 0] = (kg_loss + KG_L2_LAMBDA * kg_l2
                         + cf_loss + CF_L2_LAMBDA * cf_l2)


def _kg_map(i):
    return (i % _NBLK, 0)


def _cf_map(i):
    return (jnp.minimum(i, _NBLK - 1), 0)


_loss_call = pl.pallas_call(
    _loss_body,
    grid=(2 * _NBLK,),
    in_specs=[pl.BlockSpec((_BLK, D), _cf_map) for _ in range(5)]
    + [pl.BlockSpec((_BLK, D), _kg_map) for _ in range(4)],
    out_specs=pl.BlockSpec((1, 1), lambda i: (0, 0),
                           memory_space=pltpu.SMEM),
    out_shape=jax.ShapeDtypeStruct((1, 1), jnp.float32),
    scratch_shapes=[pltpu.SMEM((16,), jnp.float32)],
)


def kernel(user_ids, item_pos_ids, item_neg_ids, h, r, pos_t, neg_t,
           user_table, item_table, entity_table, relation_table):
    idx = [x.astype(jnp.int32) for x in
           (user_ids, item_pos_ids, item_neg_ids, h, r, pos_t, neg_t)]
    gathered = _sc_gather()(*idx, user_table, item_table, entity_table,
                            relation_table)
    out = _loss_call(*gathered)
    return jnp.reshape(out, ())
